# baseline (speedup 1.0000x reference)
import jax
import jax.numpy as jnp
from jax.experimental import pallas as pl

N = 4096
D = 512
BLOCK_M = 512


def _fused_body(a_ref, x_ref, w_ref, b_ref, o_ref):
    masked = jnp.dot(a_ref[...], x_ref[...], preferred_element_type=jnp.float32)
    out = jnp.dot(masked, w_ref[...], preferred_element_type=jnp.float32)
    o_ref[...] = jnp.maximum(out + b_ref[...], 0.0)


def kernel(x, W, b, A_tilde):
    b2 = b.reshape(1, D)
    grid = (N // BLOCK_M,)
    out = pl.pallas_call(
        _fused_body,
        grid=grid,
        in_specs=[
            pl.BlockSpec((BLOCK_M, N), lambda i: (i, 0)),
            pl.BlockSpec((N, D), lambda i: (0, 0)),
            pl.BlockSpec((D, D), lambda i: (0, 0)),
            pl.BlockSpec((1, D), lambda i: (0, 0)),
        ],
        out_specs=pl.BlockSpec((BLOCK_M, D), lambda i: (i, 0)),
        out_shape=jax.ShapeDtypeStruct((N, D), jnp.float32),
    )(A_tilde, x, W, b2)
    return out


# speedup vs baseline: 1.0932x; 1.0932x over previous
import jax
import jax.numpy as jnp
from jax.experimental import pallas as pl
from jax.experimental.pallas import tpu as pltpu

N = 4096
D = 512
BM = 512
S = N // BM
NBUF = 4
NOBUF = 2


def _body(a_hbm, x_ref, w_ref, b_ref, o_hbm, abuf, asem, obuf, osem):
    def a_copy(slot, idx):
        return pltpu.make_async_copy(
            a_hbm.at[pl.ds(idx * BM, BM), :], abuf.at[slot], asem.at[slot])

    def o_copy(slot, idx):
        return pltpu.make_async_copy(
            obuf.at[slot], o_hbm.at[pl.ds(idx * BM, BM), :], osem.at[slot])

    for k in range(NBUF):
        a_copy(k, k).start()

    def step(i, _):
        aslot = jax.lax.rem(i, NBUF)
        oslot = jax.lax.rem(i, NOBUF)
        a_copy(aslot, i).wait()

        @pl.when(i >= NOBUF)
        def _wait_out():
            o_copy(oslot, i - NOBUF).wait()

        masked = jnp.dot(abuf[aslot], x_ref[...], preferred_element_type=jnp.float32)
        out = jnp.dot(masked, w_ref[...], preferred_element_type=jnp.float32)
        obuf[oslot] = jnp.maximum(out + b_ref[...], 0.0)
        o_copy(oslot, i).start()

        @pl.when(i + NBUF < S)
        def _prefetch():
            a_copy(aslot, i + NBUF).start()

        return 0

    jax.lax.fori_loop(0, S, step, 0)

    for k in range(NOBUF):
        o_copy(k, S - NOBUF + k).wait()


def kernel(x, W, b, A_tilde):
    b2 = b.reshape(1, D)
    out = pl.pallas_call(
        _body,
        in_specs=[
            pl.BlockSpec(memory_space=pltpu.MemorySpace.HBM),
            pl.BlockSpec(memory_space=pltpu.MemorySpace.VMEM),
            pl.BlockSpec(memory_space=pltpu.MemorySpace.VMEM),
            pl.BlockSpec(memory_space=pltpu.MemorySpace.VMEM),
        ],
        out_specs=pl.BlockSpec(memory_space=pltpu.MemorySpace.HBM),
        out_shape=jax.ShapeDtypeStruct((N, D), jnp.float32),
        scratch_shapes=[
            pltpu.VMEM((NBUF, BM, N), jnp.float32),
            pltpu.SemaphoreType.DMA((NBUF,)),
            pltpu.VMEM((NOBUF, BM, D), jnp.float32),
            pltpu.SemaphoreType.DMA((NOBUF,)),
        ],
    )(A_tilde, x, W, b2)
    return out
